# baseline (device time: 34253 ns/iter reference)
import jax
import jax.numpy as jnp
from jax import lax
from jax.experimental import pallas as pl
from jax.experimental.pallas import tpu as pltpu

N_DEV = 8
M_BLK = 512
K_BLK = 512
N_OUT = 2048

F8 = jnp.float8_e4m3fn


def kernel(x, w_mat, scale_x, scale_w):
    def body(x_ref, w_hbm, sx_ref, sw_ref, out_ref,
             x8_ref, buf_ref, wbuf_ref, w8_ref, send_sems, recv_sems,
             wdma_sems):
        my = lax.axis_index("i")

        def w_copy(s, slot):
            return pltpu.make_async_copy(
                w_hbm.at[pl.ds(s * K_BLK, K_BLK), :],
                wbuf_ref.at[slot],
                wdma_sems.at[slot],
            )

        w_copy(my, 0).start()

        barrier = pltpu.get_barrier_semaphore()
        for k in range(1, N_DEV):
            pl.semaphore_signal(
                barrier, inc=1,
                device_id=((my + k) % N_DEV,),
                device_id_type=pl.DeviceIdType.MESH,
            )
        pl.semaphore_wait(barrier, N_DEV - 1)

        sends = []
        for k in range(1, N_DEV):
            e = (my + k) % N_DEV
            x8_ref[pl.ds(e * M_BLK, M_BLK), :] = (
                x_ref[pl.ds(e * M_BLK, M_BLK), :].astype(F8))
            rdma = pltpu.make_async_remote_copy(
                src_ref=x8_ref.at[pl.ds(e * M_BLK, M_BLK), :],
                dst_ref=buf_ref.at[my],
                send_sem=send_sems.at[k - 1],
                recv_sem=recv_sems.at[my],
                device_id=(e,),
                device_id_type=pl.DeviceIdType.MESH,
            )
            rdma.start()
            sends.append(rdma)
        x8_ref[pl.ds(my * M_BLK, M_BLK), :] = (
            x_ref[pl.ds(my * M_BLK, M_BLK), :].astype(F8))

        w_copy(my, 0).wait()
        w8_ref[my] = wbuf_ref[0].astype(F8)
        w_copy((my - 1) % N_DEV, 1).start()
        acc = jnp.dot(
            x8_ref[pl.ds(my * M_BLK, M_BLK), :],
            w8_ref[my],
            preferred_element_type=jnp.float32,
        )
        for j in range(1, N_DEV):
            s = (my - j) % N_DEV
            if j + 1 < N_DEV:
                w_copy((my - (j + 1)) % N_DEV, (j + 1) % 2).start()
            w_copy(s, j % 2).wait()
            w8_ref[s] = wbuf_ref[j % 2].astype(F8)
            recv = pltpu.make_async_remote_copy(
                src_ref=buf_ref.at[s],
                dst_ref=buf_ref.at[s],
                send_sem=send_sems.at[N_DEV - 1],
                recv_sem=recv_sems.at[s],
                device_id=(s,),
                device_id_type=pl.DeviceIdType.MESH,
            )
            recv.wait_recv()
            acc += jnp.dot(
                buf_ref[s], w8_ref[s], preferred_element_type=jnp.float32)

        for rdma in sends:
            rdma.wait_send()

        scale = sx_ref[0] * sw_ref[0]
        out_ref[:, :] = jnp.maximum(acc * scale, 0.0)

    return pl.pallas_call(
        body,
        out_shape=jax.ShapeDtypeStruct((M_BLK, N_OUT), jnp.float32),
        in_specs=[
            pl.BlockSpec(memory_space=pltpu.VMEM),
            pl.BlockSpec(memory_space=pl.ANY),
            pl.BlockSpec(memory_space=pltpu.SMEM),
            pl.BlockSpec(memory_space=pltpu.SMEM),
        ],
        out_specs=pl.BlockSpec(memory_space=pltpu.VMEM),
        scratch_shapes=[
            pltpu.VMEM((N_DEV * M_BLK, K_BLK), F8),
            pltpu.VMEM((N_DEV, M_BLK, K_BLK), F8),
            pltpu.VMEM((2, K_BLK, N_OUT), jnp.float32),
            pltpu.VMEM((N_DEV, K_BLK, N_OUT), F8),
            pltpu.SemaphoreType.DMA((N_DEV,)),
            pltpu.SemaphoreType.DMA((N_DEV,)),
            pltpu.SemaphoreType.DMA((2,)),
        ],
        compiler_params=pltpu.CompilerParams(
            collective_id=0, vmem_limit_bytes=100 * 1024 * 1024),
    )(x, w_mat, scale_x, scale_w)


# device time: 31102 ns/iter; 1.1013x vs baseline; 1.1013x over previous
import jax
import jax.numpy as jnp
from jax import lax
from jax.experimental import pallas as pl
from jax.experimental.pallas import tpu as pltpu

N_DEV = 8
M_BLK = 512
K_BLK = 512
N_OUT = 2048

F8 = jnp.float8_e4m3fn


def kernel(x, w_mat, scale_x, scale_w):
    def body(x_hbm, w_hbm, sx_ref, sw_ref, out_ref,
             x8_ref, buf_ref, xbuf_ref, wbuf_ref, w8_ref,
             send_sems, recv_sems, xdma_sems, wdma_sems):
        my = lax.axis_index("i")

        def x_copy(e, slot):
            return pltpu.make_async_copy(
                x_hbm.at[pl.ds(e * M_BLK, M_BLK), :],
                xbuf_ref.at[slot],
                xdma_sems.at[slot],
            )

        def w_copy(s, slot):
            return pltpu.make_async_copy(
                w_hbm.at[pl.ds(s * K_BLK, K_BLK), :],
                wbuf_ref.at[slot],
                wdma_sems.at[slot],
            )

        barrier = pltpu.get_barrier_semaphore()
        for k in range(1, N_DEV):
            pl.semaphore_signal(
                barrier, inc=1,
                device_id=((my + k) % N_DEV,),
                device_id_type=pl.DeviceIdType.MESH,
            )

        w_copy(my, 0).start()
        x_copy((my + 1) % N_DEV, 0).start()
        x_copy((my + 2) % N_DEV, 1).start()

        sends = []
        for k in range(1, N_DEV + 1):
            e = (my + k) % N_DEV
            slot = (k - 1) % 2
            x_copy(e, slot).wait()
            x8_ref[pl.ds(e * M_BLK, M_BLK), :] = xbuf_ref[slot].astype(F8)
            if k + 2 <= N_DEV:
                x_copy((my + k + 2) % N_DEV, slot).start()
            if k == 1:
                pl.semaphore_wait(barrier, N_DEV - 1)
            if k < N_DEV:
                rdma = pltpu.make_async_remote_copy(
                    src_ref=x8_ref.at[pl.ds(e * M_BLK, M_BLK), :],
                    dst_ref=buf_ref.at[my],
                    send_sem=send_sems.at[k - 1],
                    recv_sem=recv_sems.at[my],
                    device_id=(e,),
                    device_id_type=pl.DeviceIdType.MESH,
                )
                rdma.start()
                sends.append(rdma)

        w_copy(my, 0).wait()
        w8_ref[my] = wbuf_ref[0].astype(F8)
        w_copy((my - 1) % N_DEV, 1).start()
        acc = jnp.dot(
            x8_ref[pl.ds(my * M_BLK, M_BLK), :],
            w8_ref[my],
            preferred_element_type=jnp.float32,
        )
        for j in range(1, N_DEV):
            s = (my - j) % N_DEV
            if j + 1 < N_DEV:
                w_copy((my - (j + 1)) % N_DEV, (j + 1) % 2).start()
            w_copy(s, j % 2).wait()
            w8_ref[s] = wbuf_ref[j % 2].astype(F8)
            recv = pltpu.make_async_remote_copy(
                src_ref=buf_ref.at[s],
                dst_ref=buf_ref.at[s],
                send_sem=send_sems.at[N_DEV - 1],
                recv_sem=recv_sems.at[s],
                device_id=(s,),
                device_id_type=pl.DeviceIdType.MESH,
            )
            recv.wait_recv()
            acc += jnp.dot(
                buf_ref[s], w8_ref[s], preferred_element_type=jnp.float32)

        for rdma in sends:
            rdma.wait_send()

        scale = sx_ref[0] * sw_ref[0]
        out_ref[:, :] = jnp.maximum(acc * scale, 0.0)

    return pl.pallas_call(
        body,
        out_shape=jax.ShapeDtypeStruct((M_BLK, N_OUT), jnp.float32),
        in_specs=[
            pl.BlockSpec(memory_space=pl.ANY),
            pl.BlockSpec(memory_space=pl.ANY),
            pl.BlockSpec(memory_space=pltpu.SMEM),
            pl.BlockSpec(memory_space=pltpu.SMEM),
        ],
        out_specs=pl.BlockSpec(memory_space=pltpu.VMEM),
        scratch_shapes=[
            pltpu.VMEM((N_DEV * M_BLK, K_BLK), F8),
            pltpu.VMEM((N_DEV, M_BLK, K_BLK), F8),
            pltpu.VMEM((2, M_BLK, K_BLK), jnp.float32),
            pltpu.VMEM((2, K_BLK, N_OUT), jnp.float32),
            pltpu.VMEM((N_DEV, K_BLK, N_OUT), F8),
            pltpu.SemaphoreType.DMA((N_DEV,)),
            pltpu.SemaphoreType.DMA((N_DEV,)),
            pltpu.SemaphoreType.DMA((2,)),
            pltpu.SemaphoreType.DMA((2,)),
        ],
        compiler_params=pltpu.CompilerParams(
            collective_id=0, vmem_limit_bytes=100 * 1024 * 1024),
    )(x, w_mat, scale_x, scale_w)


# device time: 30789 ns/iter; 1.1125x vs baseline; 1.0102x over previous
import jax
import jax.numpy as jnp
from jax import lax
from jax.experimental import pallas as pl
from jax.experimental.pallas import tpu as pltpu

N_DEV = 8
M_BLK = 512
K_BLK = 512
N_OUT = 2048

F8 = jnp.float8_e4m3fn


def kernel(x, w_mat, scale_x, scale_w):
    def body(x_hbm, w_hbm, sx_ref, sw_ref, out_ref,
             x8_ref, buf_ref, xbuf_ref, wbuf_ref, w8_ref,
             send_sems, recv_sems, xdma_sems, wdma_sems):
        my = lax.axis_index("i")

        def x_copy(e, slot):
            return pltpu.make_async_copy(
                x_hbm.at[pl.ds(e * M_BLK, M_BLK), :],
                xbuf_ref.at[slot],
                xdma_sems.at[slot],
            )

        def w_copy(s, slot):
            return pltpu.make_async_copy(
                w_hbm.at[pl.ds(s * K_BLK, K_BLK), :],
                wbuf_ref.at[slot],
                wdma_sems.at[slot],
            )

        barrier = pltpu.get_barrier_semaphore()
        for k in range(1, N_DEV):
            pl.semaphore_signal(
                barrier, inc=1,
                device_id=((my + k) % N_DEV,),
                device_id_type=pl.DeviceIdType.MESH,
            )

        w_copy(my, 0).start()
        x_copy((my + 1) % N_DEV, 0).start()
        x_copy((my + 2) % N_DEV, 1).start()

        sends = []
        for k in range(1, N_DEV + 1):
            e = (my + k) % N_DEV
            slot = (k - 1) % 2
            x_copy(e, slot).wait()
            x8_ref[pl.ds(e * M_BLK, M_BLK), :] = xbuf_ref[slot].astype(F8)
            if k + 2 <= N_DEV:
                x_copy((my + k + 2) % N_DEV, slot).start()
            if k == 1:
                pl.semaphore_wait(barrier, N_DEV - 1)
            if k < N_DEV:
                rdma = pltpu.make_async_remote_copy(
                    src_ref=x8_ref.at[pl.ds(e * M_BLK, M_BLK), :],
                    dst_ref=buf_ref.at[my],
                    send_sem=send_sems.at[k - 1],
                    recv_sem=recv_sems.at[my],
                    device_id=(e,),
                    device_id_type=pl.DeviceIdType.MESH,
                )
                rdma.start()
                sends.append(rdma)

        w_copy(my, 0).wait()
        w8_ref[my] = wbuf_ref[0].astype(F8)
        w_copy((my - 1) % N_DEV, 1).start()
        acc = jnp.dot(
            x8_ref[pl.ds(my * M_BLK, M_BLK), :],
            w8_ref[my],
            preferred_element_type=jnp.float32,
        )
        scale = sx_ref[0] * sw_ref[0]
        H = N_OUT // 2
        for j in range(1, N_DEV):
            s = (my - j) % N_DEV
            if j + 1 < N_DEV:
                w_copy((my - (j + 1)) % N_DEV, (j + 1) % 2).start()
            w_copy(s, j % 2).wait()
            w8_ref[s] = wbuf_ref[j % 2].astype(F8)
            recv = pltpu.make_async_remote_copy(
                src_ref=buf_ref.at[s],
                dst_ref=buf_ref.at[s],
                send_sem=send_sems.at[N_DEV - 1],
                recv_sem=recv_sems.at[s],
                device_id=(s,),
                device_id_type=pl.DeviceIdType.MESH,
            )
            recv.wait_recv()
            if j < N_DEV - 1:
                acc += jnp.dot(
                    buf_ref[s], w8_ref[s], preferred_element_type=jnp.float32)
            else:
                a0 = acc[:, :H] + jnp.dot(
                    buf_ref[s], w8_ref[s, :, :H],
                    preferred_element_type=jnp.float32)
                out_ref[:, :H] = jnp.maximum(a0 * scale, 0.0)
                a1 = acc[:, H:] + jnp.dot(
                    buf_ref[s], w8_ref[s, :, H:],
                    preferred_element_type=jnp.float32)
                out_ref[:, H:] = jnp.maximum(a1 * scale, 0.0)

        for rdma in sends:
            rdma.wait_send()

    return pl.pallas_call(
        body,
        out_shape=jax.ShapeDtypeStruct((M_BLK, N_OUT), jnp.float32),
        in_specs=[
            pl.BlockSpec(memory_space=pl.ANY),
            pl.BlockSpec(memory_space=pl.ANY),
            pl.BlockSpec(memory_space=pltpu.SMEM),
            pl.BlockSpec(memory_space=pltpu.SMEM),
        ],
        out_specs=pl.BlockSpec(memory_space=pltpu.VMEM),
        scratch_shapes=[
            pltpu.VMEM((N_DEV * M_BLK, K_BLK), F8),
            pltpu.VMEM((N_DEV, M_BLK, K_BLK), F8),
            pltpu.VMEM((2, M_BLK, K_BLK), jnp.float32),
            pltpu.VMEM((2, K_BLK, N_OUT), jnp.float32),
            pltpu.VMEM((N_DEV, K_BLK, N_OUT), F8),
            pltpu.SemaphoreType.DMA((N_DEV,)),
            pltpu.SemaphoreType.DMA((N_DEV,)),
            pltpu.SemaphoreType.DMA((2,)),
            pltpu.SemaphoreType.DMA((2,)),
        ],
        compiler_params=pltpu.CompilerParams(
            collective_id=0, vmem_limit_bytes=100 * 1024 * 1024),
    )(x, w_mat, scale_x, scale_w)


# device time: 30524 ns/iter; 1.1222x vs baseline; 1.0087x over previous
import jax
import jax.numpy as jnp
from jax import lax
from jax.experimental import pallas as pl
from jax.experimental.pallas import tpu as pltpu

N_DEV = 8
M_BLK = 512
K_BLK = 512
N_OUT = 2048

F8 = jnp.float8_e4m3fn


def kernel(x, w_mat, scale_x, scale_w):
    def body(x_hbm, w_hbm, sx_ref, sw_ref, out_hbm,
             x8_ref, buf_ref, xbuf_ref, wbuf_ref, w8_ref, out_vm,
             send_sems, recv_sems, xdma_sems, wdma_sems, odma_sems):
        my = lax.axis_index("i")

        def x_copy(e, slot):
            return pltpu.make_async_copy(
                x_hbm.at[pl.ds(e * M_BLK, M_BLK), :],
                xbuf_ref.at[slot],
                xdma_sems.at[slot],
            )

        def w_copy(s, slot):
            return pltpu.make_async_copy(
                w_hbm.at[pl.ds(s * K_BLK, K_BLK), :],
                wbuf_ref.at[slot],
                wdma_sems.at[slot],
            )

        barrier = pltpu.get_barrier_semaphore()
        for k in range(1, N_DEV):
            pl.semaphore_signal(
                barrier, inc=1,
                device_id=((my + k) % N_DEV,),
                device_id_type=pl.DeviceIdType.MESH,
            )

        w_copy(my, 0).start()
        x_copy((my + 1) % N_DEV, 0).start()
        x_copy((my + 2) % N_DEV, 1).start()

        sends = []
        for k in range(1, N_DEV + 1):
            e = (my + k) % N_DEV
            slot = (k - 1) % 2
            x_copy(e, slot).wait()
            x8_ref[pl.ds(e * M_BLK, M_BLK), :] = xbuf_ref[slot].astype(F8)
            if k + 2 <= N_DEV:
                x_copy((my + k + 2) % N_DEV, slot).start()
            if k == 1:
                pl.semaphore_wait(barrier, N_DEV - 1)
            if k < N_DEV:
                rdma = pltpu.make_async_remote_copy(
                    src_ref=x8_ref.at[pl.ds(e * M_BLK, M_BLK), :],
                    dst_ref=buf_ref.at[my],
                    send_sem=send_sems.at[k - 1],
                    recv_sem=recv_sems.at[my],
                    device_id=(e,),
                    device_id_type=pl.DeviceIdType.MESH,
                )
                rdma.start()
                sends.append(rdma)

        w_copy(my, 0).wait()
        w8_ref[my] = wbuf_ref[0].astype(F8)
        w_copy((my - 1) % N_DEV, 1).start()
        acc = jnp.dot(
            x8_ref[pl.ds(my * M_BLK, M_BLK), :],
            w8_ref[my],
            preferred_element_type=jnp.float32,
        )
        scale = sx_ref[0] * sw_ref[0]
        H = N_OUT // 2
        for j in range(1, N_DEV):
            s = (my - j) % N_DEV
            if j + 1 < N_DEV:
                w_copy((my - (j + 1)) % N_DEV, (j + 1) % 2).start()
            w_copy(s, j % 2).wait()
            w8_ref[s] = wbuf_ref[j % 2].astype(F8)
            recv = pltpu.make_async_remote_copy(
                src_ref=buf_ref.at[s],
                dst_ref=buf_ref.at[s],
                send_sem=send_sems.at[N_DEV - 1],
                recv_sem=recv_sems.at[s],
                device_id=(s,),
                device_id_type=pl.DeviceIdType.MESH,
            )
            recv.wait_recv()
            if j < N_DEV - 1:
                acc += jnp.dot(
                    buf_ref[s], w8_ref[s], preferred_element_type=jnp.float32)
            else:
                a0 = acc[:, :H] + jnp.dot(
                    buf_ref[s], w8_ref[s, :, :H],
                    preferred_element_type=jnp.float32)
                out_vm[0] = jnp.maximum(a0 * scale, 0.0)
                o0 = pltpu.make_async_copy(
                    out_vm.at[0], out_hbm.at[:, pl.ds(0, H)], odma_sems.at[0])
                o0.start()
                a1 = acc[:, H:] + jnp.dot(
                    buf_ref[s], w8_ref[s, :, H:],
                    preferred_element_type=jnp.float32)
                out_vm[1] = jnp.maximum(a1 * scale, 0.0)
                o1 = pltpu.make_async_copy(
                    out_vm.at[1], out_hbm.at[:, pl.ds(H, H)], odma_sems.at[1])
                o1.start()
                o0.wait()
                o1.wait()

        for rdma in sends:
            rdma.wait_send()

    return pl.pallas_call(
        body,
        out_shape=jax.ShapeDtypeStruct((M_BLK, N_OUT), jnp.float32),
        in_specs=[
            pl.BlockSpec(memory_space=pl.ANY),
            pl.BlockSpec(memory_space=pl.ANY),
            pl.BlockSpec(memory_space=pltpu.SMEM),
            pl.BlockSpec(memory_space=pltpu.SMEM),
        ],
        out_specs=pl.BlockSpec(memory_space=pl.ANY),
        scratch_shapes=[
            pltpu.VMEM((N_DEV * M_BLK, K_BLK), F8),
            pltpu.VMEM((N_DEV, M_BLK, K_BLK), F8),
            pltpu.VMEM((2, M_BLK, K_BLK), jnp.float32),
            pltpu.VMEM((2, K_BLK, N_OUT), jnp.float32),
            pltpu.VMEM((N_DEV, K_BLK, N_OUT), F8),
            pltpu.VMEM((2, M_BLK, N_OUT // 2), jnp.float32),
            pltpu.SemaphoreType.DMA((N_DEV,)),
            pltpu.SemaphoreType.DMA((N_DEV,)),
            pltpu.SemaphoreType.DMA((2,)),
            pltpu.SemaphoreType.DMA((2,)),
            pltpu.SemaphoreType.DMA((2,)),
        ],
        compiler_params=pltpu.CompilerParams(
            collective_id=0, vmem_limit_bytes=100 * 1024 * 1024),
    )(x, w_mat, scale_x, scale_w)


# device time: 30523 ns/iter; 1.1222x vs baseline; 1.0000x over previous
import jax
import jax.numpy as jnp
from jax import lax
from jax.experimental import pallas as pl
from jax.experimental.pallas import tpu as pltpu

N_DEV = 8
M_BLK = 512
K_BLK = 512
N_OUT = 2048

F8 = jnp.float8_e4m3fn


def kernel(x, w_mat, scale_x, scale_w):
    def body(x_hbm, w_hbm, sx_ref, sw_ref, out_hbm,
             x8_ref, buf_ref, xbuf_ref, wbuf_ref, w8_ref, out_vm,
             send_sems, recv_sems, xdma_sems, wdma_sems, odma_sems,
             ready_sems):
        my = lax.axis_index("i")

        def x_copy(e, slot):
            return pltpu.make_async_copy(
                x_hbm.at[pl.ds(e * M_BLK, M_BLK), :],
                xbuf_ref.at[slot],
                xdma_sems.at[slot],
            )

        def w_copy(s, slot):
            return pltpu.make_async_copy(
                w_hbm.at[pl.ds(s * K_BLK, K_BLK), :],
                wbuf_ref.at[slot],
                wdma_sems.at[slot],
            )

        for k in range(1, N_DEV):
            pl.semaphore_signal(
                ready_sems.at[my], inc=1,
                device_id=((my + k) % N_DEV,),
                device_id_type=pl.DeviceIdType.MESH,
            )
        barrier = pltpu.get_barrier_semaphore()
        pl.semaphore_signal(barrier, inc=1, device_id=(my,),
                            device_id_type=pl.DeviceIdType.MESH)
        pl.semaphore_wait(barrier, 1)

        w_copy(my, 0).start()
        x_copy((my + 1) % N_DEV, 0).start()
        x_copy((my + 2) % N_DEV, 1).start()

        sends = []
        for k in range(1, N_DEV + 1):
            e = (my + k) % N_DEV
            slot = (k - 1) % 2
            x_copy(e, slot).wait()
            x8_ref[pl.ds(e * M_BLK, M_BLK), :] = xbuf_ref[slot].astype(F8)
            if k + 2 <= N_DEV:
                x_copy((my + k + 2) % N_DEV, slot).start()
            if k < N_DEV:
                pl.semaphore_wait(ready_sems.at[e], 1)
                rdma = pltpu.make_async_remote_copy(
                    src_ref=x8_ref.at[pl.ds(e * M_BLK, M_BLK), :],
                    dst_ref=buf_ref.at[my],
                    send_sem=send_sems.at[k - 1],
                    recv_sem=recv_sems.at[my],
                    device_id=(e,),
                    device_id_type=pl.DeviceIdType.MESH,
                )
                rdma.start()
                sends.append(rdma)

        w_copy(my, 0).wait()
        w8_ref[my] = wbuf_ref[0].astype(F8)
        w_copy((my - 1) % N_DEV, 1).start()
        acc = jnp.dot(
            x8_ref[pl.ds(my * M_BLK, M_BLK), :],
            w8_ref[my],
            preferred_element_type=jnp.float32,
        )
        scale = sx_ref[0] * sw_ref[0]
        H = N_OUT // 2
        for j in range(1, N_DEV):
            s = (my - j) % N_DEV
            if j + 1 < N_DEV:
                w_copy((my - (j + 1)) % N_DEV, (j + 1) % 2).start()
            w_copy(s, j % 2).wait()
            w8_ref[s] = wbuf_ref[j % 2].astype(F8)
            recv = pltpu.make_async_remote_copy(
                src_ref=buf_ref.at[s],
                dst_ref=buf_ref.at[s],
                send_sem=send_sems.at[N_DEV - 1],
                recv_sem=recv_sems.at[s],
                device_id=(s,),
                device_id_type=pl.DeviceIdType.MESH,
            )
            recv.wait_recv()
            if j < N_DEV - 1:
                acc += jnp.dot(
                    buf_ref[s], w8_ref[s], preferred_element_type=jnp.float32)
            else:
                a0 = acc[:, :H] + jnp.dot(
                    buf_ref[s], w8_ref[s, :, :H],
                    preferred_element_type=jnp.float32)
                out_vm[0] = jnp.maximum(a0 * scale, 0.0)
                o0 = pltpu.make_async_copy(
                    out_vm.at[0], out_hbm.at[:, pl.ds(0, H)], odma_sems.at[0])
                o0.start()
                a1 = acc[:, H:] + jnp.dot(
                    buf_ref[s], w8_ref[s, :, H:],
                    preferred_element_type=jnp.float32)
                out_vm[1] = jnp.maximum(a1 * scale, 0.0)
                o1 = pltpu.make_async_copy(
                    out_vm.at[1], out_hbm.at[:, pl.ds(H, H)], odma_sems.at[1])
                o1.start()
                o0.wait()
                o1.wait()

        for rdma in sends:
            rdma.wait_send()

    return pl.pallas_call(
        body,
        out_shape=jax.ShapeDtypeStruct((M_BLK, N_OUT), jnp.float32),
        in_specs=[
            pl.BlockSpec(memory_space=pl.ANY),
            pl.BlockSpec(memory_space=pl.ANY),
            pl.BlockSpec(memory_space=pltpu.SMEM),
            pl.BlockSpec(memory_space=pltpu.SMEM),
        ],
        out_specs=pl.BlockSpec(memory_space=pl.ANY),
        scratch_shapes=[
            pltpu.VMEM((N_DEV * M_BLK, K_BLK), F8),
            pltpu.VMEM((N_DEV, M_BLK, K_BLK), F8),
            pltpu.VMEM((2, M_BLK, K_BLK), jnp.float32),
            pltpu.VMEM((2, K_BLK, N_OUT), jnp.float32),
            pltpu.VMEM((N_DEV, K_BLK, N_OUT), F8),
            pltpu.VMEM((2, M_BLK, N_OUT // 2), jnp.float32),
            pltpu.SemaphoreType.DMA((N_DEV,)),
            pltpu.SemaphoreType.DMA((N_DEV,)),
            pltpu.SemaphoreType.DMA((2,)),
            pltpu.SemaphoreType.DMA((2,)),
            pltpu.SemaphoreType.DMA((2,)),
            pltpu.SemaphoreType.REGULAR((N_DEV,)),
        ],
        compiler_params=pltpu.CompilerParams(
            collective_id=0, vmem_limit_bytes=100 * 1024 * 1024),
    )(x, w_mat, scale_x, scale_w)
